# Initial kernel scaffold; baseline (speedup 1.0000x reference)
import jax
import jax.numpy as jnp
from jax.experimental import pallas as pl


def kernel(x, edge_index, Wl1, bl1, Wr1, Wl2, bl2, Wr2, Wl3, bl3, Wr3, Cw1, Cb1, Cw2, Cb2):
    raise NotImplementedError("write your pallas kernel here")


# speedup vs baseline: 2.9126x; 2.9126x over previous
import functools

import jax
import jax.numpy as jnp
from jax import lax
from jax.experimental import pallas as pl
from jax.experimental.pallas import tpu as pltpu
from jax.experimental.pallas import tpu_sc as plsc

N_NODES = 10000
DIM_IN = 128
DIM_H = 256
N_EDGES = 320000

E_PAD = ((N_EDGES + 32767) // 32768) * 32768
E_ROWS = E_PAD // 128
ROWS_L1 = E_ROWS // 2 // 16
ROWS_L23 = E_ROWS // 16
N_TAB = 10112
ZSTRIPE = N_TAB // 16

_MESH = plsc.VectorSubcoreMesh(core_axis_name="c", subcore_axis_name="s")


def _fill_zeros(ref, nrows, width):
    g = width // 16

    def body(k, _):
        i = k // g
        j = k % g
        ref[i, pl.ds(j * 16, 16)] = jnp.zeros((16,), jnp.float32)
        return 0

    lax.fori_loop(0, nrows * g, body, 0)


IDX_BATCH = 8


def _zero_table(zsrc, table, r0):
    def body(k, _):
        pltpu.sync_copy(zsrc, table.at[pl.ds(r0 + k * 8, 8)])
        return 0

    lax.fori_loop(0, ZSTRIPE // 8, body, 0)


def _copy_out(table, out, sr, o0):
    def body(k, _):
        pltpu.sync_copy(table.at[pl.ds(sr + k * 128, 128)],
                        out.at[pl.ds(o0 + k * 128, 128)])
        return 0

    lax.fori_loop(0, 4, body, 0)
    pltpu.sync_copy(table.at[pl.ds(sr + 512, ZSTRIPE - 512)],
                    out.at[pl.ds(o0 + 512, ZSTRIPE - 512)])


@functools.partial(
    pl.kernel,
    out_type=jax.ShapeDtypeStruct((2 * N_TAB, 128), jnp.float32),
    mesh=_MESH,
    scratch_types=[
        pltpu.VMEM_SHARED((N_TAB, 128), jnp.float32),
        pltpu.VMEM((IDX_BATCH, 128), jnp.int32),
        pltpu.VMEM((128, 128), jnp.float32),
        pltpu.VMEM((8, 128), jnp.float32),
    ],
)
def _sc_cnt(dst_hbm, cnt_out, table_sh, dst_v, ones_v, zeros_v):
    c = lax.axis_index("c")
    s = lax.axis_index("s")

    _fill_zeros(zeros_v, 8, 128)

    def fill_ones(k, _):
        i = k // 8
        j = k % 8
        ones_v[i, pl.ds(j * 16, 16)] = jnp.ones((16,), jnp.float32)
        return 0

    lax.fori_loop(0, 1024, fill_ones, 0)

    r0 = s * ZSTRIPE
    _zero_table(zeros_v, table_sh, r0)
    plsc.subcore_barrier()

    row0 = c * (E_ROWS // 2) + s * ROWS_L1

    def batch(b, _):
        pltpu.sync_copy(dst_hbm.at[pl.ds(row0 + b * IDX_BATCH, IDX_BATCH)],
                        dst_v)

        def edge(j, _):
            pltpu.sync_copy(ones_v, table_sh.at[dst_v.at[j]], add=True)
            return 0

        lax.fori_loop(0, IDX_BATCH, edge, 0)
        return 0

    lax.fori_loop(0, ROWS_L1 // IDX_BATCH, batch, 0)
    plsc.subcore_barrier()
    _copy_out(table_sh, cnt_out, r0, c * N_TAB + r0)


@functools.partial(
    pl.kernel,
    out_type=jax.ShapeDtypeStruct((2 * N_TAB, 128), jnp.float32),
    mesh=_MESH,
    scratch_types=[
        pltpu.VMEM_SHARED((N_TAB, 128), jnp.float32),
        pltpu.VMEM((IDX_BATCH, 128), jnp.int32),
        pltpu.VMEM((IDX_BATCH, 128), jnp.int32),
        pltpu.VMEM((128, 128), jnp.float32),
        pltpu.VMEM((8, 128), jnp.float32),
        pltpu.SemaphoreType.DMA,
    ],
)
def _sc_agg_l1(x_hbm, src_hbm, dst_hbm, agg_out,
               table_sh, src_v, dst_v, rows_v, zeros_v, sem):
    c = lax.axis_index("c")
    s = lax.axis_index("s")

    _fill_zeros(zeros_v, 8, 128)
    r0 = s * ZSTRIPE
    _zero_table(zeros_v, table_sh, r0)
    plsc.subcore_barrier()

    row0 = c * (E_ROWS // 2) + s * ROWS_L1

    def batch(b, _):
        pltpu.sync_copy(src_hbm.at[pl.ds(row0 + b * IDX_BATCH, IDX_BATCH)],
                        src_v)
        pltpu.sync_copy(dst_hbm.at[pl.ds(row0 + b * IDX_BATCH, IDX_BATCH)],
                        dst_v)

        def edge(j, _):
            pltpu.async_copy(x_hbm.at[src_v.at[j]], rows_v, sem).wait()
            pltpu.sync_copy(rows_v, table_sh.at[dst_v.at[j]], add=True)
            return 0

        lax.fori_loop(0, IDX_BATCH, edge, 0)
        return 0

    lax.fori_loop(0, ROWS_L1 // IDX_BATCH, batch, 0)
    plsc.subcore_barrier()
    _copy_out(table_sh, agg_out, r0, c * N_TAB + r0)


@functools.partial(
    pl.kernel,
    out_type=jax.ShapeDtypeStruct((2 * N_TAB, 128), jnp.float32),
    mesh=_MESH,
    scratch_types=[
        pltpu.VMEM_SHARED((N_TAB, 128), jnp.float32),
        pltpu.VMEM((IDX_BATCH, 128), jnp.int32),
        pltpu.VMEM((IDX_BATCH, 128), jnp.int32),
        pltpu.VMEM((128, 128), jnp.float32),
        pltpu.VMEM((8, 128), jnp.float32),
        pltpu.SemaphoreType.DMA,
    ],
)
def _sc_agg_l23(feat0_hbm, feat1_hbm, src_hbm, dst_hbm, agg_out,
                table_sh, src_v, dst_v, rows_v, zeros_v, sem):
    c = lax.axis_index("c")
    s = lax.axis_index("s")

    _fill_zeros(zeros_v, 8, 128)
    r0 = s * ZSTRIPE
    _zero_table(zeros_v, table_sh, r0)
    plsc.subcore_barrier()

    row0 = s * ROWS_L23

    def run(feat):
        def batch(b, _):
            pltpu.sync_copy(
                src_hbm.at[pl.ds(row0 + b * IDX_BATCH, IDX_BATCH)], src_v)
            pltpu.sync_copy(
                dst_hbm.at[pl.ds(row0 + b * IDX_BATCH, IDX_BATCH)], dst_v)

            def edge(j, _):
                pltpu.async_copy(feat.at[src_v.at[j]], rows_v, sem).wait()
                pltpu.sync_copy(rows_v, table_sh.at[dst_v.at[j]], add=True)
                return 0

            lax.fori_loop(0, IDX_BATCH, edge, 0)
            return 0

        lax.fori_loop(0, ROWS_L23 // IDX_BATCH, batch, 0)

    @pl.when(c == 0)
    def _():
        run(feat0_hbm)

    @pl.when(c == 1)
    def _():
        run(feat1_hbm)

    plsc.subcore_barrier()

    sr = s * ZSTRIPE
    _copy_out(table_sh, agg_out, sr, c * N_TAB + sr)


def _dot_t(a, w):
    return lax.dot_general(a, w, (((1,), (1,)), ((), ())),
                           precision=lax.Precision.HIGHEST,
                           preferred_element_type=jnp.float32)


_ROWS_BLK = 1000
_N_BLKS = N_NODES // _ROWS_BLK


def _tc1_body(agg0, agg1, cnt0, cnt1, x, wl, bl, wr, h0, h1):
    cnt = cnt0[:, 0:1] + cnt1[:, 0:1]
    inv = 1.0 / jnp.maximum(cnt, 1.0)
    mean = (agg0[...] + agg1[...]) * inv
    out = _dot_t(mean, wl[...]) + _dot_t(x[...], wr[...]) + bl[...]
    nrm = jnp.sqrt(jnp.sum(out * out, axis=1, keepdims=True))
    out = out / jnp.maximum(nrm, 1e-12)
    h0[...] = out[:, :128]
    h1[...] = out[:, 128:]


def _tc23_body(agg0, agg1, cnt0, cnt1, x0, x1, wla, wlb, bl, wra, wrb,
               h0, h1):
    cnt = cnt0[:, 0:1] + cnt1[:, 0:1]
    inv = 1.0 / jnp.maximum(cnt, 1.0)
    out = (_dot_t(agg0[...] * inv, wla[...]) + _dot_t(agg1[...] * inv, wlb[...])
           + _dot_t(x0[...], wra[...]) + _dot_t(x1[...], wrb[...]) + bl[...])
    nrm = jnp.sqrt(jnp.sum(out * out, axis=1, keepdims=True))
    out = out / jnp.maximum(nrm, 1e-12)
    h0[...] = out[:, :128]
    h1[...] = out[:, 128:]


def _tc3_body(agg0, agg1, cnt0, cnt1, x0, x1, wla, wlb, bl, wra, wrb,
              cw1, cb1, cw2, cb2, res, acc):
    i = pl.program_id(0)
    cnt = cnt0[:, 0:1] + cnt1[:, 0:1]
    inv = 1.0 / jnp.maximum(cnt, 1.0)
    out = (_dot_t(agg0[...] * inv, wla[...]) + _dot_t(agg1[...] * inv, wlb[...])
           + _dot_t(x0[...], wra[...]) + _dot_t(x1[...], wrb[...]) + bl[...])
    nrm = jnp.sqrt(jnp.sum(out * out, axis=1, keepdims=True))
    out = out / jnp.maximum(nrm, 1e-12)

    @pl.when(i == 0)
    def _():
        acc[...] = jnp.zeros_like(acc)

    acc[...] += jnp.sum(out, axis=0, keepdims=True)

    @pl.when(i == _N_BLKS - 1)
    def _():
        g = acc[...] * (1.0 / N_NODES)
        z = jnp.maximum(_dot_t(g, cw1[...]) + cb1[...], 0.0)
        res[...] = jnp.sum(z * cw2[...], axis=1, keepdims=True) + cb2[...]


def _row_spec(w):
    return pl.BlockSpec((_ROWS_BLK, w), lambda i: (i, 0))


def _full_spec(r, c):
    return pl.BlockSpec((r, c), lambda i: (0, 0))


def kernel(x, edge_index, Wl1, bl1, Wr1, Wl2, bl2, Wr2, Wl3, bl3, Wr3,
           Cw1, Cb1, Cw2, Cb2):
    src = edge_index[0]
    dst = edge_index[1]
    pad = E_PAD - N_EDGES
    srcp = jnp.concatenate(
        [src, jnp.zeros((pad,), jnp.int32)]).reshape(E_ROWS, 128)
    dstp = jnp.concatenate(
        [dst, jnp.full((pad,), N_NODES, jnp.int32)]).reshape(E_ROWS, 128)

    bl1r = bl1[None, :]
    bl2r = bl2[None, :]
    bl3r = bl3[None, :]
    cb1r = Cb1[None, :]
    cb2r = Cb2[None, :]

    cntp = _sc_cnt(dstp)
    cnt0, cnt1 = cntp[:N_NODES], cntp[N_TAB:N_TAB + N_NODES]

    aggp = _sc_agg_l1(x, srcp, dstp)
    agg0, agg1 = aggp[:N_NODES], aggp[N_TAB:N_TAB + N_NODES]

    h0, h1 = pl.pallas_call(
        _tc1_body,
        grid=(_N_BLKS,),
        in_specs=[
            _row_spec(128), _row_spec(128), _row_spec(128), _row_spec(128),
            _row_spec(128), _full_spec(256, 128), _full_spec(1, 256),
            _full_spec(256, 128),
        ],
        out_specs=[_row_spec(128), _row_spec(128)],
        out_shape=[jax.ShapeDtypeStruct((N_NODES, 128), jnp.float32)] * 2,
    )(agg0, agg1, cnt0, cnt1, x, Wl1, bl1r, Wr1)

    aggp = _sc_agg_l23(h0, h1, srcp, dstp)
    agg0, agg1 = aggp[:N_NODES], aggp[N_TAB:N_TAB + N_NODES]
    h0, h1 = pl.pallas_call(
        _tc23_body,
        grid=(_N_BLKS,),
        in_specs=[
            _row_spec(128), _row_spec(128), _row_spec(128), _row_spec(128),
            _row_spec(128), _row_spec(128),
            _full_spec(256, 128), _full_spec(256, 128), _full_spec(1, 256),
            _full_spec(256, 128), _full_spec(256, 128),
        ],
        out_specs=[_row_spec(128), _row_spec(128)],
        out_shape=[jax.ShapeDtypeStruct((N_NODES, 128), jnp.float32)] * 2,
    )(agg0, agg1, cnt0, cnt1, h0, h1,
      Wl2[:, :128], Wl2[:, 128:], bl2r, Wr2[:, :128], Wr2[:, 128:])

    aggp = _sc_agg_l23(h0, h1, srcp, dstp)
    agg0, agg1 = aggp[:N_NODES], aggp[N_TAB:N_TAB + N_NODES]
    res = pl.pallas_call(
        _tc3_body,
        grid=(_N_BLKS,),
        in_specs=[
            _row_spec(128), _row_spec(128), _row_spec(128), _row_spec(128),
            _row_spec(128), _row_spec(128),
            _full_spec(256, 128), _full_spec(256, 128), _full_spec(1, 256),
            _full_spec(256, 128), _full_spec(256, 128),
            _full_spec(256, 256), _full_spec(1, 256), _full_spec(1, 256),
            _full_spec(1, 1),
        ],
        out_specs=pl.BlockSpec((1, 1), lambda i: (0, 0)),
        out_shape=jax.ShapeDtypeStruct((1, 1), jnp.float32),
        scratch_shapes=[pltpu.VMEM((1, 256), jnp.float32)],
        compiler_params=pltpu.CompilerParams(
            dimension_semantics=("arbitrary",)),
    )(agg0, agg1, cnt0, cnt1, h0, h1,
      Wl3[:, :128], Wl3[:, 128:], bl3r, Wr3[:, :128], Wr3[:, 128:],
      Cw1, cb1r, Cw2, cb2r)
    return res


# speedup vs baseline: 3.2471x; 1.1149x over previous
import functools

import jax
import jax.numpy as jnp
from jax import lax
from jax.experimental import pallas as pl
from jax.experimental.pallas import tpu as pltpu
from jax.experimental.pallas import tpu_sc as plsc

N_NODES = 10000
DIM_IN = 128
DIM_H = 256
N_EDGES = 320000

E_PAD = ((N_EDGES + 32767) // 32768) * 32768
E_ROWS = E_PAD // 128
ROWS_L1 = E_ROWS // 2 // 16
ROWS_L23 = E_ROWS // 16
N_TAB = 10112
ZSTRIPE = N_TAB // 16

_MESH = plsc.VectorSubcoreMesh(core_axis_name="c", subcore_axis_name="s")


def _fill_zeros(ref, nrows, width):
    g = width // 16

    def body(k, _):
        i = k // g
        j = k % g
        ref[i, pl.ds(j * 16, 16)] = jnp.zeros((16,), jnp.float32)
        return 0

    lax.fori_loop(0, nrows * g, body, 0)


IDX_BATCH = 8


def _zero_table(zsrc, table, r0):
    def body(k, _):
        pltpu.sync_copy(zsrc, table.at[pl.ds(r0 + k * 8, 8)])
        return 0

    lax.fori_loop(0, ZSTRIPE // 8, body, 0)


def _copy_out(table, out, sr, o0):
    def body(k, _):
        pltpu.sync_copy(table.at[pl.ds(sr + k * 128, 128)],
                        out.at[pl.ds(o0 + k * 128, 128)])
        return 0

    lax.fori_loop(0, 4, body, 0)
    pltpu.sync_copy(table.at[pl.ds(sr + 512, ZSTRIPE - 512)],
                    out.at[pl.ds(o0 + 512, ZSTRIPE - 512)])


def _edge_pipeline(feat, src_hbm, dst_hbm, table_sh, row0, nrows,
                   src_a, dst_a, src_b, dst_b, rows0, rows1,
                   sem_g, sem_s, sem_i):
    npair = nrows // IDX_BATCH // 2

    pltpu.sync_copy(src_hbm.at[pl.ds(row0, IDX_BATCH)], src_a)
    pltpu.sync_copy(dst_hbm.at[pl.ds(row0, IDX_BATCH)], dst_a)

    def process(src_x, dst_x):
        g = pltpu.async_copy(feat.at[src_x.at[0]], rows0, sem_g)
        for j in range(IDX_BATCH):
            cur = rows0 if j % 2 == 0 else rows1
            nxt = rows1 if j % 2 == 0 else rows0
            g.wait()
            sd = pltpu.async_copy(cur, table_sh.at[dst_x.at[j]], sem_s,
                                  add=True)
            if j < IDX_BATCH - 1:
                g = pltpu.async_copy(feat.at[src_x.at[j + 1]], nxt, sem_g)
            sd.wait()

    def pair(p, _):
        base1 = row0 + (2 * p + 1) * IDX_BATCH
        base2 = jnp.minimum(base1 + IDX_BATCH, E_ROWS - IDX_BATCH)
        i1 = pltpu.async_copy(src_hbm.at[pl.ds(base1, IDX_BATCH)], src_b,
                              sem_i)
        i2 = pltpu.async_copy(dst_hbm.at[pl.ds(base1, IDX_BATCH)], dst_b,
                              sem_i)
        process(src_a, dst_a)
        i1.wait()
        i2.wait()
        i3 = pltpu.async_copy(src_hbm.at[pl.ds(base2, IDX_BATCH)], src_a,
                              sem_i)
        i4 = pltpu.async_copy(dst_hbm.at[pl.ds(base2, IDX_BATCH)], dst_a,
                              sem_i)
        process(src_b, dst_b)
        i3.wait()
        i4.wait()
        return 0

    lax.fori_loop(0, npair, pair, 0)


_AGG_SCRATCH = [
    pltpu.VMEM_SHARED((N_TAB, 128), jnp.float32),
    pltpu.VMEM((IDX_BATCH, 128), jnp.int32),
    pltpu.VMEM((IDX_BATCH, 128), jnp.int32),
    pltpu.VMEM((IDX_BATCH, 128), jnp.int32),
    pltpu.VMEM((IDX_BATCH, 128), jnp.int32),
    pltpu.VMEM((128, 128), jnp.float32),
    pltpu.VMEM((128, 128), jnp.float32),
    pltpu.VMEM((8, 128), jnp.float32),
    pltpu.SemaphoreType.DMA,
    pltpu.SemaphoreType.DMA,
    pltpu.SemaphoreType.DMA,
]


@functools.partial(
    pl.kernel,
    out_type=jax.ShapeDtypeStruct((2 * N_TAB, 128), jnp.float32),
    mesh=_MESH,
    scratch_types=[
        pltpu.VMEM_SHARED((N_TAB, 128), jnp.float32),
        pltpu.VMEM((IDX_BATCH, 128), jnp.int32),
        pltpu.VMEM((128, 128), jnp.float32),
        pltpu.VMEM((8, 128), jnp.float32),
        pltpu.SemaphoreType.DMA,
    ],
)
def _sc_cnt(dst_hbm, cnt_out, table_sh, dst_v, ones_v, zeros_v, sem_s):
    c = lax.axis_index("c")
    s = lax.axis_index("s")

    _fill_zeros(zeros_v, 8, 128)

    def fill_ones(k, _):
        i = k // 8
        j = k % 8
        ones_v[i, pl.ds(j * 16, 16)] = jnp.ones((16,), jnp.float32)
        return 0

    lax.fori_loop(0, 1024, fill_ones, 0)

    r0 = s * ZSTRIPE
    _zero_table(zeros_v, table_sh, r0)
    plsc.subcore_barrier()

    row0 = c * (E_ROWS // 2) + s * ROWS_L1

    def batch(b, _):
        pltpu.sync_copy(dst_hbm.at[pl.ds(row0 + b * IDX_BATCH, IDX_BATCH)],
                        dst_v)
        descs = [
            pltpu.async_copy(ones_v, table_sh.at[dst_v.at[j]], sem_s,
                             add=True)
            for j in range(IDX_BATCH)
        ]
        for d in descs:
            d.wait()
        return 0

    lax.fori_loop(0, ROWS_L1 // IDX_BATCH, batch, 0)
    plsc.subcore_barrier()
    _copy_out(table_sh, cnt_out, r0, c * N_TAB + r0)


@functools.partial(
    pl.kernel,
    out_type=jax.ShapeDtypeStruct((2 * N_TAB, 128), jnp.float32),
    mesh=_MESH,
    scratch_types=list(_AGG_SCRATCH),
)
def _sc_agg_l1(x_hbm, src_hbm, dst_hbm, agg_out,
               table_sh, src_a, dst_a, src_b, dst_b, rows0, rows1,
               zeros_v, sem_g, sem_s, sem_i):
    c = lax.axis_index("c")
    s = lax.axis_index("s")

    _fill_zeros(zeros_v, 8, 128)
    r0 = s * ZSTRIPE
    _zero_table(zeros_v, table_sh, r0)
    plsc.subcore_barrier()

    row0 = c * (E_ROWS // 2) + s * ROWS_L1
    _edge_pipeline(x_hbm, src_hbm, dst_hbm, table_sh, row0, ROWS_L1,
                   src_a, dst_a, src_b, dst_b, rows0, rows1,
                   sem_g, sem_s, sem_i)
    plsc.subcore_barrier()
    _copy_out(table_sh, agg_out, r0, c * N_TAB + r0)


@functools.partial(
    pl.kernel,
    out_type=jax.ShapeDtypeStruct((2 * N_TAB, 128), jnp.float32),
    mesh=_MESH,
    scratch_types=list(_AGG_SCRATCH),
)
def _sc_agg_l23(feat0_hbm, feat1_hbm, src_hbm, dst_hbm, agg_out,
                table_sh, src_a, dst_a, src_b, dst_b, rows0, rows1,
                zeros_v, sem_g, sem_s, sem_i):
    c = lax.axis_index("c")
    s = lax.axis_index("s")

    _fill_zeros(zeros_v, 8, 128)
    r0 = s * ZSTRIPE
    _zero_table(zeros_v, table_sh, r0)
    plsc.subcore_barrier()

    row0 = s * ROWS_L23

    def run(feat):
        _edge_pipeline(feat, src_hbm, dst_hbm, table_sh, row0, ROWS_L23,
                       src_a, dst_a, src_b, dst_b, rows0, rows1,
                       sem_g, sem_s, sem_i)

    @pl.when(c == 0)
    def _():
        run(feat0_hbm)

    @pl.when(c == 1)
    def _():
        run(feat1_hbm)

    plsc.subcore_barrier()

    sr = s * ZSTRIPE
    _copy_out(table_sh, agg_out, sr, c * N_TAB + sr)


def _dot_t(a, w):
    return lax.dot_general(a, w, (((1,), (1,)), ((), ())),
                           precision=lax.Precision.HIGHEST,
                           preferred_element_type=jnp.float32)


_ROWS_BLK = 1000
_N_BLKS = N_NODES // _ROWS_BLK


def _tc1_body(agg0, agg1, cnt0, cnt1, x, wl, bl, wr, h0, h1):
    cnt = cnt0[:, 0:1] + cnt1[:, 0:1]
    inv = 1.0 / jnp.maximum(cnt, 1.0)
    mean = (agg0[...] + agg1[...]) * inv
    out = _dot_t(mean, wl[...]) + _dot_t(x[...], wr[...]) + bl[...]
    nrm = jnp.sqrt(jnp.sum(out * out, axis=1, keepdims=True))
    out = out / jnp.maximum(nrm, 1e-12)
    h0[...] = out[:, :128]
    h1[...] = out[:, 128:]


def _tc23_body(agg0, agg1, cnt0, cnt1, x0, x1, wla, wlb, bl, wra, wrb,
               h0, h1):
    cnt = cnt0[:, 0:1] + cnt1[:, 0:1]
    inv = 1.0 / jnp.maximum(cnt, 1.0)
    out = (_dot_t(agg0[...] * inv, wla[...]) + _dot_t(agg1[...] * inv, wlb[...])
           + _dot_t(x0[...], wra[...]) + _dot_t(x1[...], wrb[...]) + bl[...])
    nrm = jnp.sqrt(jnp.sum(out * out, axis=1, keepdims=True))
    out = out / jnp.maximum(nrm, 1e-12)
    h0[...] = out[:, :128]
    h1[...] = out[:, 128:]


def _tc3_body(agg0, agg1, cnt0, cnt1, x0, x1, wla, wlb, bl, wra, wrb,
              cw1, cb1, cw2, cb2, res, acc):
    i = pl.program_id(0)
    cnt = cnt0[:, 0:1] + cnt1[:, 0:1]
    inv = 1.0 / jnp.maximum(cnt, 1.0)
    out = (_dot_t(agg0[...] * inv, wla[...]) + _dot_t(agg1[...] * inv, wlb[...])
           + _dot_t(x0[...], wra[...]) + _dot_t(x1[...], wrb[...]) + bl[...])
    nrm = jnp.sqrt(jnp.sum(out * out, axis=1, keepdims=True))
    out = out / jnp.maximum(nrm, 1e-12)

    @pl.when(i == 0)
    def _():
        acc[...] = jnp.zeros_like(acc)

    acc[...] += jnp.sum(out, axis=0, keepdims=True)

    @pl.when(i == _N_BLKS - 1)
    def _():
        g = acc[...] * (1.0 / N_NODES)
        z = jnp.maximum(_dot_t(g, cw1[...]) + cb1[...], 0.0)
        res[...] = jnp.sum(z * cw2[...], axis=1, keepdims=True) + cb2[...]


def _row_spec(w):
    return pl.BlockSpec((_ROWS_BLK, w), lambda i: (i, 0))


def _full_spec(r, c):
    return pl.BlockSpec((r, c), lambda i: (0, 0))


def kernel(x, edge_index, Wl1, bl1, Wr1, Wl2, bl2, Wr2, Wl3, bl3, Wr3,
           Cw1, Cb1, Cw2, Cb2):
    src = edge_index[0]
    dst = edge_index[1]
    pad = E_PAD - N_EDGES
    srcp = jnp.concatenate(
        [src, jnp.zeros((pad,), jnp.int32)]).reshape(E_ROWS, 128)
    dstp = jnp.concatenate(
        [dst, jnp.full((pad,), N_NODES, jnp.int32)]).reshape(E_ROWS, 128)

    bl1r = bl1[None, :]
    bl2r = bl2[None, :]
    bl3r = bl3[None, :]
    cb1r = Cb1[None, :]
    cb2r = Cb2[None, :]

    cntp = _sc_cnt(dstp)
    cnt0, cnt1 = cntp[:N_NODES], cntp[N_TAB:N_TAB + N_NODES]

    aggp = _sc_agg_l1(x, srcp, dstp)
    agg0, agg1 = aggp[:N_NODES], aggp[N_TAB:N_TAB + N_NODES]

    h0, h1 = pl.pallas_call(
        _tc1_body,
        grid=(_N_BLKS,),
        in_specs=[
            _row_spec(128), _row_spec(128), _row_spec(128), _row_spec(128),
            _row_spec(128), _full_spec(256, 128), _full_spec(1, 256),
            _full_spec(256, 128),
        ],
        out_specs=[_row_spec(128), _row_spec(128)],
        out_shape=[jax.ShapeDtypeStruct((N_NODES, 128), jnp.float32)] * 2,
    )(agg0, agg1, cnt0, cnt1, x, Wl1, bl1r, Wr1)

    aggp = _sc_agg_l23(h0, h1, srcp, dstp)
    agg0, agg1 = aggp[:N_NODES], aggp[N_TAB:N_TAB + N_NODES]
    h0, h1 = pl.pallas_call(
        _tc23_body,
        grid=(_N_BLKS,),
        in_specs=[
            _row_spec(128), _row_spec(128), _row_spec(128), _row_spec(128),
            _row_spec(128), _row_spec(128),
            _full_spec(256, 128), _full_spec(256, 128), _full_spec(1, 256),
            _full_spec(256, 128), _full_spec(256, 128),
        ],
        out_specs=[_row_spec(128), _row_spec(128)],
        out_shape=[jax.ShapeDtypeStruct((N_NODES, 128), jnp.float32)] * 2,
    )(agg0, agg1, cnt0, cnt1, h0, h1,
      Wl2[:, :128], Wl2[:, 128:], bl2r, Wr2[:, :128], Wr2[:, 128:])

    aggp = _sc_agg_l23(h0, h1, srcp, dstp)
    agg0, agg1 = aggp[:N_NODES], aggp[N_TAB:N_TAB + N_NODES]
    res = pl.pallas_call(
        _tc3_body,
        grid=(_N_BLKS,),
        in_specs=[
            _row_spec(128), _row_spec(128), _row_spec(128), _row_spec(128),
            _row_spec(128), _row_spec(128),
            _full_spec(256, 128), _full_spec(256, 128), _full_spec(1, 256),
            _full_spec(256, 128), _full_spec(256, 128),
            _full_spec(256, 256), _full_spec(1, 256), _full_spec(1, 256),
            _full_spec(1, 1),
        ],
        out_specs=pl.BlockSpec((1, 1), lambda i: (0, 0)),
        out_shape=jax.ShapeDtypeStruct((1, 1), jnp.float32),
        scratch_shapes=[pltpu.VMEM((1, 256), jnp.float32)],
        compiler_params=pltpu.CompilerParams(
            dimension_semantics=("arbitrary",)),
    )(agg0, agg1, cnt0, cnt1, h0, h1,
      Wl3[:, :128], Wl3[:, 128:], bl3r, Wr3[:, :128], Wr3[:, 128:],
      Cw1, cb1r, Cw2, cb2r)
    return res


# speedup vs baseline: 3.2544x; 1.0022x over previous
import functools

import jax
import jax.numpy as jnp
from jax import lax
from jax.experimental import pallas as pl
from jax.experimental.pallas import tpu as pltpu
from jax.experimental.pallas import tpu_sc as plsc

N_NODES = 10000
DIM_IN = 128
DIM_H = 256
N_EDGES = 320000

E_PAD = ((N_EDGES + 32767) // 32768) * 32768
E_ROWS = E_PAD // 128
ROWS_L1 = E_ROWS // 2 // 16
ROWS_L23 = E_ROWS // 16
N_TAB = 10112
ZSTRIPE = N_TAB // 16

_MESH = plsc.VectorSubcoreMesh(core_axis_name="c", subcore_axis_name="s")


def _fill_zeros(ref, nrows, width):
    g = width // 16

    def body(k, _):
        i = k // g
        j = k % g
        ref[i, pl.ds(j * 16, 16)] = jnp.zeros((16,), jnp.float32)
        return 0

    lax.fori_loop(0, nrows * g, body, 0)


IDX_BATCH = 8


def _zero_table(zsrc, table, r0, sem):
    descs = [
        pltpu.async_copy(zsrc, table.at[pl.ds(r0 + k * 128, 128)], sem)
        for k in range(4)
    ]
    descs.append(
        pltpu.async_copy(zsrc.at[pl.ds(0, ZSTRIPE - 512)],
                         table.at[pl.ds(r0 + 512, ZSTRIPE - 512)], sem))
    for d in descs:
        d.wait()


def _copy_out(table, out, sr, o0, sem):
    descs = [
        pltpu.async_copy(table.at[pl.ds(sr + k * 128, 128)],
                         out.at[pl.ds(o0 + k * 128, 128)], sem)
        for k in range(4)
    ]
    descs.append(
        pltpu.async_copy(table.at[pl.ds(sr + 512, ZSTRIPE - 512)],
                         out.at[pl.ds(o0 + 512, ZSTRIPE - 512)], sem))
    for d in descs:
        d.wait()


def _edge_pipeline(feat, src_hbm, dst_hbm, table_sh, row0, nrows,
                   src_a, dst_a, src_b, dst_b, rows0, rows1,
                   sem_g, sem_s, sem_i):
    npair = nrows // IDX_BATCH // 2

    pltpu.sync_copy(src_hbm.at[pl.ds(row0, IDX_BATCH)], src_a)
    pltpu.sync_copy(dst_hbm.at[pl.ds(row0, IDX_BATCH)], dst_a)

    def process(src_x, dst_x):
        g = pltpu.async_copy(feat.at[src_x.at[0]], rows0, sem_g)
        for j in range(IDX_BATCH):
            cur = rows0 if j % 2 == 0 else rows1
            nxt = rows1 if j % 2 == 0 else rows0
            g.wait()
            sd = pltpu.async_copy(cur, table_sh.at[dst_x.at[j]], sem_s,
                                  add=True)
            if j < IDX_BATCH - 1:
                g = pltpu.async_copy(feat.at[src_x.at[j + 1]], nxt, sem_g)
            sd.wait()

    def pair(p, _):
        base1 = row0 + (2 * p + 1) * IDX_BATCH
        base2 = jnp.minimum(base1 + IDX_BATCH, E_ROWS - IDX_BATCH)
        i1 = pltpu.async_copy(src_hbm.at[pl.ds(base1, IDX_BATCH)], src_b,
                              sem_i)
        i2 = pltpu.async_copy(dst_hbm.at[pl.ds(base1, IDX_BATCH)], dst_b,
                              sem_i)
        process(src_a, dst_a)
        i1.wait()
        i2.wait()
        i3 = pltpu.async_copy(src_hbm.at[pl.ds(base2, IDX_BATCH)], src_a,
                              sem_i)
        i4 = pltpu.async_copy(dst_hbm.at[pl.ds(base2, IDX_BATCH)], dst_a,
                              sem_i)
        process(src_b, dst_b)
        i3.wait()
        i4.wait()
        return 0

    lax.fori_loop(0, npair, pair, 0)


_AGG_SCRATCH = [
    pltpu.VMEM_SHARED((N_TAB, 128), jnp.float32),
    pltpu.VMEM((IDX_BATCH, 128), jnp.int32),
    pltpu.VMEM((IDX_BATCH, 128), jnp.int32),
    pltpu.VMEM((IDX_BATCH, 128), jnp.int32),
    pltpu.VMEM((IDX_BATCH, 128), jnp.int32),
    pltpu.VMEM((128, 128), jnp.float32),
    pltpu.VMEM((128, 128), jnp.float32),
    pltpu.SemaphoreType.DMA,
    pltpu.SemaphoreType.DMA,
    pltpu.SemaphoreType.DMA,
]


@functools.partial(
    pl.kernel,
    out_type=jax.ShapeDtypeStruct((2 * N_TAB, 128), jnp.float32),
    mesh=_MESH,
    scratch_types=[
        pltpu.VMEM_SHARED((N_TAB, 128), jnp.float32),
        pltpu.VMEM((IDX_BATCH, 128), jnp.int32),
        pltpu.VMEM((128, 128), jnp.float32),
        pltpu.SemaphoreType.DMA,
    ],
)
def _sc_cnt(dst_hbm, cnt_out, table_sh, dst_v, ones_v, sem_s):
    c = lax.axis_index("c")
    s = lax.axis_index("s")

    _fill_zeros(ones_v, 128, 128)
    r0 = s * ZSTRIPE
    _zero_table(ones_v, table_sh, r0, sem_s)

    def fill_ones(k, _):
        i = k // 8
        j = k % 8
        ones_v[i, pl.ds(j * 16, 16)] = jnp.ones((16,), jnp.float32)
        return 0

    lax.fori_loop(0, 1024, fill_ones, 0)
    plsc.subcore_barrier()

    row0 = c * (E_ROWS // 2) + s * ROWS_L1

    def batch(b, _):
        pltpu.sync_copy(dst_hbm.at[pl.ds(row0 + b * IDX_BATCH, IDX_BATCH)],
                        dst_v)
        descs = [
            pltpu.async_copy(ones_v, table_sh.at[dst_v.at[j]], sem_s,
                             add=True)
            for j in range(IDX_BATCH)
        ]
        for d in descs:
            d.wait()
        return 0

    lax.fori_loop(0, ROWS_L1 // IDX_BATCH, batch, 0)
    plsc.subcore_barrier()
    _copy_out(table_sh, cnt_out, r0, c * N_TAB + r0, sem_s)


@functools.partial(
    pl.kernel,
    out_type=jax.ShapeDtypeStruct((2 * N_TAB, 128), jnp.float32),
    mesh=_MESH,
    scratch_types=list(_AGG_SCRATCH),
)
def _sc_agg_l1(x_hbm, src_hbm, dst_hbm, agg_out,
               table_sh, src_a, dst_a, src_b, dst_b, rows0, rows1,
               sem_g, sem_s, sem_i):
    c = lax.axis_index("c")
    s = lax.axis_index("s")

    _fill_zeros(rows0, 128, 128)
    r0 = s * ZSTRIPE
    _zero_table(rows0, table_sh, r0, sem_g)
    plsc.subcore_barrier()

    row0 = c * (E_ROWS // 2) + s * ROWS_L1
    _edge_pipeline(x_hbm, src_hbm, dst_hbm, table_sh, row0, ROWS_L1,
                   src_a, dst_a, src_b, dst_b, rows0, rows1,
                   sem_g, sem_s, sem_i)
    plsc.subcore_barrier()
    _copy_out(table_sh, agg_out, r0, c * N_TAB + r0, sem_g)


@functools.partial(
    pl.kernel,
    out_type=jax.ShapeDtypeStruct((2 * N_TAB, 128), jnp.float32),
    mesh=_MESH,
    scratch_types=list(_AGG_SCRATCH),
)
def _sc_agg_l23(feat0_hbm, feat1_hbm, src_hbm, dst_hbm, agg_out,
                table_sh, src_a, dst_a, src_b, dst_b, rows0, rows1,
                sem_g, sem_s, sem_i):
    c = lax.axis_index("c")
    s = lax.axis_index("s")

    _fill_zeros(rows0, 128, 128)
    r0 = s * ZSTRIPE
    _zero_table(rows0, table_sh, r0, sem_g)
    plsc.subcore_barrier()

    row0 = s * ROWS_L23

    def run(feat):
        _edge_pipeline(feat, src_hbm, dst_hbm, table_sh, row0, ROWS_L23,
                       src_a, dst_a, src_b, dst_b, rows0, rows1,
                       sem_g, sem_s, sem_i)

    @pl.when(c == 0)
    def _():
        run(feat0_hbm)

    @pl.when(c == 1)
    def _():
        run(feat1_hbm)

    plsc.subcore_barrier()

    sr = s * ZSTRIPE
    _copy_out(table_sh, agg_out, sr, c * N_TAB + sr, sem_g)


def _dot_t(a, w):
    return lax.dot_general(a, w, (((1,), (1,)), ((), ())),
                           precision=lax.Precision.HIGHEST,
                           preferred_element_type=jnp.float32)


_ROWS_BLK = 1000
_N_BLKS = N_NODES // _ROWS_BLK


def _tc1_body(agg0, agg1, cnt0, cnt1, x, wl, bl, wr, h0, h1):
    cnt = cnt0[:, 0:1] + cnt1[:, 0:1]
    inv = 1.0 / jnp.maximum(cnt, 1.0)
    mean = (agg0[...] + agg1[...]) * inv
    out = _dot_t(mean, wl[...]) + _dot_t(x[...], wr[...]) + bl[...]
    nrm = jnp.sqrt(jnp.sum(out * out, axis=1, keepdims=True))
    out = out / jnp.maximum(nrm, 1e-12)
    h0[...] = out[:, :128]
    h1[...] = out[:, 128:]


def _tc23_body(agg0, agg1, cnt0, cnt1, x0, x1, wla, wlb, bl, wra, wrb,
               h0, h1):
    cnt = cnt0[:, 0:1] + cnt1[:, 0:1]
    inv = 1.0 / jnp.maximum(cnt, 1.0)
    out = (_dot_t(agg0[...] * inv, wla[...]) + _dot_t(agg1[...] * inv, wlb[...])
           + _dot_t(x0[...], wra[...]) + _dot_t(x1[...], wrb[...]) + bl[...])
    nrm = jnp.sqrt(jnp.sum(out * out, axis=1, keepdims=True))
    out = out / jnp.maximum(nrm, 1e-12)
    h0[...] = out[:, :128]
    h1[...] = out[:, 128:]


def _tc3_body(agg0, agg1, cnt0, cnt1, x0, x1, wla, wlb, bl, wra, wrb,
              cw1, cb1, cw2, cb2, res, acc):
    i = pl.program_id(0)
    cnt = cnt0[:, 0:1] + cnt1[:, 0:1]
    inv = 1.0 / jnp.maximum(cnt, 1.0)
    out = (_dot_t(agg0[...] * inv, wla[...]) + _dot_t(agg1[...] * inv, wlb[...])
           + _dot_t(x0[...], wra[...]) + _dot_t(x1[...], wrb[...]) + bl[...])
    nrm = jnp.sqrt(jnp.sum(out * out, axis=1, keepdims=True))
    out = out / jnp.maximum(nrm, 1e-12)

    @pl.when(i == 0)
    def _():
        acc[...] = jnp.zeros_like(acc)

    acc[...] += jnp.sum(out, axis=0, keepdims=True)

    @pl.when(i == _N_BLKS - 1)
    def _():
        g = acc[...] * (1.0 / N_NODES)
        z = jnp.maximum(_dot_t(g, cw1[...]) + cb1[...], 0.0)
        res[...] = jnp.sum(z * cw2[...], axis=1, keepdims=True) + cb2[...]


def _row_spec(w):
    return pl.BlockSpec((_ROWS_BLK, w), lambda i: (i, 0))


def _full_spec(r, c):
    return pl.BlockSpec((r, c), lambda i: (0, 0))


def kernel(x, edge_index, Wl1, bl1, Wr1, Wl2, bl2, Wr2, Wl3, bl3, Wr3,
           Cw1, Cb1, Cw2, Cb2):
    src = edge_index[0]
    dst = edge_index[1]
    pad = E_PAD - N_EDGES
    srcp = jnp.concatenate(
        [src, jnp.zeros((pad,), jnp.int32)]).reshape(E_ROWS, 128)
    dstp = jnp.concatenate(
        [dst, jnp.full((pad,), N_NODES, jnp.int32)]).reshape(E_ROWS, 128)

    bl1r = bl1[None, :]
    bl2r = bl2[None, :]
    bl3r = bl3[None, :]
    cb1r = Cb1[None, :]
    cb2r = Cb2[None, :]

    cntp = _sc_cnt(dstp)
    cnt0, cnt1 = cntp[:N_NODES], cntp[N_TAB:N_TAB + N_NODES]

    aggp = _sc_agg_l1(x, srcp, dstp)
    agg0, agg1 = aggp[:N_NODES], aggp[N_TAB:N_TAB + N_NODES]

    h0, h1 = pl.pallas_call(
        _tc1_body,
        grid=(_N_BLKS,),
        in_specs=[
            _row_spec(128), _row_spec(128), _row_spec(128), _row_spec(128),
            _row_spec(128), _full_spec(256, 128), _full_spec(1, 256),
            _full_spec(256, 128),
        ],
        out_specs=[_row_spec(128), _row_spec(128)],
        out_shape=[jax.ShapeDtypeStruct((N_NODES, 128), jnp.float32)] * 2,
    )(agg0, agg1, cnt0, cnt1, x, Wl1, bl1r, Wr1)

    aggp = _sc_agg_l23(h0, h1, srcp, dstp)
    agg0, agg1 = aggp[:N_NODES], aggp[N_TAB:N_TAB + N_NODES]
    h0, h1 = pl.pallas_call(
        _tc23_body,
        grid=(_N_BLKS,),
        in_specs=[
            _row_spec(128), _row_spec(128), _row_spec(128), _row_spec(128),
            _row_spec(128), _row_spec(128),
            _full_spec(256, 128), _full_spec(256, 128), _full_spec(1, 256),
            _full_spec(256, 128), _full_spec(256, 128),
        ],
        out_specs=[_row_spec(128), _row_spec(128)],
        out_shape=[jax.ShapeDtypeStruct((N_NODES, 128), jnp.float32)] * 2,
    )(agg0, agg1, cnt0, cnt1, h0, h1,
      Wl2[:, :128], Wl2[:, 128:], bl2r, Wr2[:, :128], Wr2[:, 128:])

    aggp = _sc_agg_l23(h0, h1, srcp, dstp)
    agg0, agg1 = aggp[:N_NODES], aggp[N_TAB:N_TAB + N_NODES]
    res = pl.pallas_call(
        _tc3_body,
        grid=(_N_BLKS,),
        in_specs=[
            _row_spec(128), _row_spec(128), _row_spec(128), _row_spec(128),
            _row_spec(128), _row_spec(128),
            _full_spec(256, 128), _full_spec(256, 128), _full_spec(1, 256),
            _full_spec(256, 128), _full_spec(256, 128),
            _full_spec(256, 256), _full_spec(1, 256), _full_spec(1, 256),
            _full_spec(1, 1),
        ],
        out_specs=pl.BlockSpec((1, 1), lambda i: (0, 0)),
        out_shape=jax.ShapeDtypeStruct((1, 1), jnp.float32),
        scratch_shapes=[pltpu.VMEM((1, 256), jnp.float32)],
        compiler_params=pltpu.CompilerParams(
            dimension_semantics=("arbitrary",)),
    )(agg0, agg1, cnt0, cnt1, h0, h1,
      Wl3[:, :128], Wl3[:, 128:], bl3r, Wr3[:, :128], Wr3[:, 128:],
      Cw1, cb1r, Cw2, cb2r)
    return res


# speedup vs baseline: 3.3963x; 1.0436x over previous
import functools

import jax
import jax.numpy as jnp
from jax import lax
from jax.experimental import pallas as pl
from jax.experimental.pallas import tpu as pltpu
from jax.experimental.pallas import tpu_sc as plsc

N_NODES = 10000
DIM_IN = 128
DIM_H = 256
N_EDGES = 320000

E_PAD = ((N_EDGES + 32767) // 32768) * 32768
E_ROWS = E_PAD // 128
ROWS_L1 = E_ROWS // 2 // 16
ROWS_L23 = E_ROWS // 16
N_TAB = 10112
ZSTRIPE = N_TAB // 16

_MESH = plsc.VectorSubcoreMesh(core_axis_name="c", subcore_axis_name="s")


def _fill_zeros(ref, nrows, width):
    g = width // 16

    def body(k, _):
        i = k // g
        j = k % g
        ref[i, pl.ds(j * 16, 16)] = jnp.zeros((16,), jnp.float32)
        return 0

    lax.fori_loop(0, nrows * g, body, 0)


IDX_BATCH = 8


def _zero_table(zsrc, table, r0, sem):
    descs = [
        pltpu.async_copy(zsrc, table.at[pl.ds(r0 + k * 128, 128)], sem)
        for k in range(4)
    ]
    descs.append(
        pltpu.async_copy(zsrc.at[pl.ds(0, ZSTRIPE - 512)],
                         table.at[pl.ds(r0 + 512, ZSTRIPE - 512)], sem))
    for d in descs:
        d.wait()


def _copy_out(table, out, sr, o0, sem):
    descs = [
        pltpu.async_copy(table.at[pl.ds(sr + k * 128, 128)],
                         out.at[pl.ds(o0 + k * 128, 128)], sem)
        for k in range(4)
    ]
    descs.append(
        pltpu.async_copy(table.at[pl.ds(sr + 512, ZSTRIPE - 512)],
                         out.at[pl.ds(o0 + 512, ZSTRIPE - 512)], sem))
    for d in descs:
        d.wait()


def _edge_pipeline(feat, src_hbm, dst_hbm, table_sh, row0, nrows,
                   src_a, dst_a, src_b, dst_b, rows0, rows1,
                   sem_g0, sem_g1, sem_s, sem_i):
    npair = nrows // IDX_BATCH // 2
    bufs = (rows0, rows1)
    sems = (sem_g0, sem_g1)

    pltpu.sync_copy(src_hbm.at[pl.ds(row0, IDX_BATCH)], src_a)
    pltpu.sync_copy(dst_hbm.at[pl.ds(row0, IDX_BATCH)], dst_a)
    pltpu.async_copy(feat.at[src_a.at[0]], rows0, sem_g0)
    pltpu.async_copy(feat.at[src_a.at[1]], rows1, sem_g1)

    def proc(cur_src, cur_dst, nxt_src, nxt_dst):
        for j in range(IDX_BATCH):
            buf, sem = bufs[j % 2], sems[j % 2]
            pltpu.make_async_copy(feat.at[pl.ds(0, 128)], buf, sem).wait()
            pltpu.async_copy(buf, table_sh.at[cur_dst.at[j]], sem_s,
                             add=True).wait()
            if j == IDX_BATCH - 3:
                pltpu.make_async_copy(src_hbm.at[pl.ds(0, IDX_BATCH)],
                                      nxt_src, sem_i).wait()
                pltpu.make_async_copy(dst_hbm.at[pl.ds(0, IDX_BATCH)],
                                      nxt_dst, sem_i).wait()
            if j < IDX_BATCH - 2:
                pltpu.async_copy(feat.at[cur_src.at[j + 2]], buf, sem)
            else:
                pltpu.async_copy(feat.at[nxt_src.at[j + 2 - IDX_BATCH]],
                                 buf, sem)

    def pair(p, _):
        base1 = row0 + (2 * p + 1) * IDX_BATCH
        base2 = jnp.minimum(base1 + IDX_BATCH, E_ROWS - IDX_BATCH)
        pltpu.async_copy(src_hbm.at[pl.ds(base1, IDX_BATCH)], src_b, sem_i)
        pltpu.async_copy(dst_hbm.at[pl.ds(base1, IDX_BATCH)], dst_b, sem_i)
        proc(src_a, dst_a, src_b, dst_b)
        pltpu.async_copy(src_hbm.at[pl.ds(base2, IDX_BATCH)], src_a, sem_i)
        pltpu.async_copy(dst_hbm.at[pl.ds(base2, IDX_BATCH)], dst_a, sem_i)
        proc(src_b, dst_b, src_a, dst_a)
        return 0

    lax.fori_loop(0, npair, pair, 0)
    pltpu.make_async_copy(feat.at[pl.ds(0, 128)], rows0, sem_g0).wait()
    pltpu.make_async_copy(feat.at[pl.ds(0, 128)], rows1, sem_g1).wait()


_AGG_SCRATCH = [
    pltpu.VMEM_SHARED((N_TAB, 128), jnp.float32),
    pltpu.VMEM((IDX_BATCH, 128), jnp.int32),
    pltpu.VMEM((IDX_BATCH, 128), jnp.int32),
    pltpu.VMEM((IDX_BATCH, 128), jnp.int32),
    pltpu.VMEM((IDX_BATCH, 128), jnp.int32),
    pltpu.VMEM((128, 128), jnp.float32),
    pltpu.VMEM((128, 128), jnp.float32),
    pltpu.SemaphoreType.DMA,
    pltpu.SemaphoreType.DMA,
    pltpu.SemaphoreType.DMA,
    pltpu.SemaphoreType.DMA,
]


@functools.partial(
    pl.kernel,
    out_type=jax.ShapeDtypeStruct((2 * N_TAB, 128), jnp.float32),
    mesh=_MESH,
    scratch_types=[
        pltpu.VMEM_SHARED((N_TAB, 128), jnp.float32),
        pltpu.VMEM((IDX_BATCH, 128), jnp.int32),
        pltpu.VMEM((128, 128), jnp.float32),
        pltpu.SemaphoreType.DMA,
    ],
)
def _sc_cnt(dst_hbm, cnt_out, table_sh, dst_v, ones_v, sem_s):
    c = lax.axis_index("c")
    s = lax.axis_index("s")

    _fill_zeros(ones_v, 128, 128)
    r0 = s * ZSTRIPE
    _zero_table(ones_v, table_sh, r0, sem_s)

    def fill_ones(k, _):
        i = k // 8
        j = k % 8
        ones_v[i, pl.ds(j * 16, 16)] = jnp.ones((16,), jnp.float32)
        return 0

    lax.fori_loop(0, 1024, fill_ones, 0)
    plsc.subcore_barrier()

    row0 = c * (E_ROWS // 2) + s * ROWS_L1

    def batch(b, _):
        pltpu.sync_copy(dst_hbm.at[pl.ds(row0 + b * IDX_BATCH, IDX_BATCH)],
                        dst_v)
        descs = [
            pltpu.async_copy(ones_v, table_sh.at[dst_v.at[j]], sem_s,
                             add=True)
            for j in range(IDX_BATCH)
        ]
        for d in descs:
            d.wait()
        return 0

    lax.fori_loop(0, ROWS_L1 // IDX_BATCH, batch, 0)
    plsc.subcore_barrier()
    _copy_out(table_sh, cnt_out, r0, c * N_TAB + r0, sem_s)


@functools.partial(
    pl.kernel,
    out_type=jax.ShapeDtypeStruct((2 * N_TAB, 128), jnp.float32),
    mesh=_MESH,
    scratch_types=list(_AGG_SCRATCH),
)
def _sc_agg_l1(x_hbm, src_hbm, dst_hbm, agg_out,
               table_sh, src_a, dst_a, src_b, dst_b, rows0, rows1,
               sem_g0, sem_g1, sem_s, sem_i):
    c = lax.axis_index("c")
    s = lax.axis_index("s")

    _fill_zeros(rows0, 128, 128)
    r0 = s * ZSTRIPE
    _zero_table(rows0, table_sh, r0, sem_g0)
    plsc.subcore_barrier()

    row0 = c * (E_ROWS // 2) + s * ROWS_L1
    _edge_pipeline(x_hbm, src_hbm, dst_hbm, table_sh, row0, ROWS_L1,
                   src_a, dst_a, src_b, dst_b, rows0, rows1,
                   sem_g0, sem_g1, sem_s, sem_i)
    plsc.subcore_barrier()
    _copy_out(table_sh, agg_out, r0, c * N_TAB + r0, sem_g0)


@functools.partial(
    pl.kernel,
    out_type=jax.ShapeDtypeStruct((2 * N_TAB, 128), jnp.float32),
    mesh=_MESH,
    scratch_types=list(_AGG_SCRATCH),
)
def _sc_agg_l23(feat0_hbm, feat1_hbm, src_hbm, dst_hbm, agg_out,
                table_sh, src_a, dst_a, src_b, dst_b, rows0, rows1,
                sem_g0, sem_g1, sem_s, sem_i):
    c = lax.axis_index("c")
    s = lax.axis_index("s")

    _fill_zeros(rows0, 128, 128)
    r0 = s * ZSTRIPE
    _zero_table(rows0, table_sh, r0, sem_g0)
    plsc.subcore_barrier()

    row0 = s * ROWS_L23

    def run(feat):
        _edge_pipeline(feat, src_hbm, dst_hbm, table_sh, row0, ROWS_L23,
                       src_a, dst_a, src_b, dst_b, rows0, rows1,
                       sem_g0, sem_g1, sem_s, sem_i)

    @pl.when(c == 0)
    def _():
        run(feat0_hbm)

    @pl.when(c == 1)
    def _():
        run(feat1_hbm)

    plsc.subcore_barrier()

    sr = s * ZSTRIPE
    _copy_out(table_sh, agg_out, sr, c * N_TAB + sr, sem_g0)


def _dot_t(a, w):
    return lax.dot_general(a, w, (((1,), (1,)), ((), ())),
                           precision=lax.Precision.HIGHEST,
                           preferred_element_type=jnp.float32)


_ROWS_BLK = 1000
_N_BLKS = N_NODES // _ROWS_BLK


def _tc1_body(agg0, agg1, cnt0, cnt1, x, wl, bl, wr, h0, h1):
    cnt = cnt0[:, 0:1] + cnt1[:, 0:1]
    inv = 1.0 / jnp.maximum(cnt, 1.0)
    mean = (agg0[...] + agg1[...]) * inv
    out = _dot_t(mean, wl[...]) + _dot_t(x[...], wr[...]) + bl[...]
    nrm = jnp.sqrt(jnp.sum(out * out, axis=1, keepdims=True))
    out = out / jnp.maximum(nrm, 1e-12)
    h0[...] = out[:, :128]
    h1[...] = out[:, 128:]


def _tc23_body(agg0, agg1, cnt0, cnt1, x0, x1, wla, wlb, bl, wra, wrb,
               h0, h1):
    cnt = cnt0[:, 0:1] + cnt1[:, 0:1]
    inv = 1.0 / jnp.maximum(cnt, 1.0)
    out = (_dot_t(agg0[...] * inv, wla[...]) + _dot_t(agg1[...] * inv, wlb[...])
           + _dot_t(x0[...], wra[...]) + _dot_t(x1[...], wrb[...]) + bl[...])
    nrm = jnp.sqrt(jnp.sum(out * out, axis=1, keepdims=True))
    out = out / jnp.maximum(nrm, 1e-12)
    h0[...] = out[:, :128]
    h1[...] = out[:, 128:]


def _tc3_body(agg0, agg1, cnt0, cnt1, x0, x1, wla, wlb, bl, wra, wrb,
              cw1, cb1, cw2, cb2, res, acc):
    i = pl.program_id(0)
    cnt = cnt0[:, 0:1] + cnt1[:, 0:1]
    inv = 1.0 / jnp.maximum(cnt, 1.0)
    out = (_dot_t(agg0[...] * inv, wla[...]) + _dot_t(agg1[...] * inv, wlb[...])
           + _dot_t(x0[...], wra[...]) + _dot_t(x1[...], wrb[...]) + bl[...])
    nrm = jnp.sqrt(jnp.sum(out * out, axis=1, keepdims=True))
    out = out / jnp.maximum(nrm, 1e-12)

    @pl.when(i == 0)
    def _():
        acc[...] = jnp.zeros_like(acc)

    acc[...] += jnp.sum(out, axis=0, keepdims=True)

    @pl.when(i == _N_BLKS - 1)
    def _():
        g = acc[...] * (1.0 / N_NODES)
        z = jnp.maximum(_dot_t(g, cw1[...]) + cb1[...], 0.0)
        res[...] = jnp.sum(z * cw2[...], axis=1, keepdims=True) + cb2[...]


def _row_spec(w):
    return pl.BlockSpec((_ROWS_BLK, w), lambda i: (i, 0))


def _full_spec(r, c):
    return pl.BlockSpec((r, c), lambda i: (0, 0))


def kernel(x, edge_index, Wl1, bl1, Wr1, Wl2, bl2, Wr2, Wl3, bl3, Wr3,
           Cw1, Cb1, Cw2, Cb2):
    src = edge_index[0]
    dst = edge_index[1]
    pad = E_PAD - N_EDGES
    srcp = jnp.concatenate(
        [src, jnp.zeros((pad,), jnp.int32)]).reshape(E_ROWS, 128)
    dstp = jnp.concatenate(
        [dst, jnp.full((pad,), N_NODES, jnp.int32)]).reshape(E_ROWS, 128)

    bl1r = bl1[None, :]
    bl2r = bl2[None, :]
    bl3r = bl3[None, :]
    cb1r = Cb1[None, :]
    cb2r = Cb2[None, :]

    cntp = _sc_cnt(dstp)
    cnt0, cnt1 = cntp[:N_NODES], cntp[N_TAB:N_TAB + N_NODES]

    aggp = _sc_agg_l1(x, srcp, dstp)
    agg0, agg1 = aggp[:N_NODES], aggp[N_TAB:N_TAB + N_NODES]

    h0, h1 = pl.pallas_call(
        _tc1_body,
        grid=(_N_BLKS,),
        in_specs=[
            _row_spec(128), _row_spec(128), _row_spec(128), _row_spec(128),
            _row_spec(128), _full_spec(256, 128), _full_spec(1, 256),
            _full_spec(256, 128),
        ],
        out_specs=[_row_spec(128), _row_spec(128)],
        out_shape=[jax.ShapeDtypeStruct((N_NODES, 128), jnp.float32)] * 2,
    )(agg0, agg1, cnt0, cnt1, x, Wl1, bl1r, Wr1)

    aggp = _sc_agg_l23(h0, h1, srcp, dstp)
    agg0, agg1 = aggp[:N_NODES], aggp[N_TAB:N_TAB + N_NODES]
    h0, h1 = pl.pallas_call(
        _tc23_body,
        grid=(_N_BLKS,),
        in_specs=[
            _row_spec(128), _row_spec(128), _row_spec(128), _row_spec(128),
            _row_spec(128), _row_spec(128),
            _full_spec(256, 128), _full_spec(256, 128), _full_spec(1, 256),
            _full_spec(256, 128), _full_spec(256, 128),
        ],
        out_specs=[_row_spec(128), _row_spec(128)],
        out_shape=[jax.ShapeDtypeStruct((N_NODES, 128), jnp.float32)] * 2,
    )(agg0, agg1, cnt0, cnt1, h0, h1,
      Wl2[:, :128], Wl2[:, 128:], bl2r, Wr2[:, :128], Wr2[:, 128:])

    aggp = _sc_agg_l23(h0, h1, srcp, dstp)
    agg0, agg1 = aggp[:N_NODES], aggp[N_TAB:N_TAB + N_NODES]
    res = pl.pallas_call(
        _tc3_body,
        grid=(_N_BLKS,),
        in_specs=[
            _row_spec(128), _row_spec(128), _row_spec(128), _row_spec(128),
            _row_spec(128), _row_spec(128),
            _full_spec(256, 128), _full_spec(256, 128), _full_spec(1, 256),
            _full_spec(256, 128), _full_spec(256, 128),
            _full_spec(256, 256), _full_spec(1, 256), _full_spec(1, 256),
            _full_spec(1, 1),
        ],
        out_specs=pl.BlockSpec((1, 1), lambda i: (0, 0)),
        out_shape=jax.ShapeDtypeStruct((1, 1), jnp.float32),
        scratch_shapes=[pltpu.VMEM((1, 256), jnp.float32)],
        compiler_params=pltpu.CompilerParams(
            dimension_semantics=("arbitrary",)),
    )(agg0, agg1, cnt0, cnt1, h0, h1,
      Wl3[:, :128], Wl3[:, 128:], bl3r, Wr3[:, :128], Wr3[:, 128:],
      Cw1, cb1r, Cw2, cb2r)
    return res


# speedup vs baseline: 3.4917x; 1.0281x over previous
import functools

import jax
import jax.numpy as jnp
from jax import lax
from jax.experimental import pallas as pl
from jax.experimental.pallas import tpu as pltpu
from jax.experimental.pallas import tpu_sc as plsc

N_NODES = 10000
DIM_IN = 128
DIM_H = 256
N_EDGES = 320000

E_PAD = ((N_EDGES + 32767) // 32768) * 32768
E_ROWS = E_PAD // 128
ROWS_L1 = E_ROWS // 2 // 16
ROWS_L23 = E_ROWS // 16
N_TAB = 10112
ZSTRIPE = N_TAB // 16

_MESH = plsc.VectorSubcoreMesh(core_axis_name="c", subcore_axis_name="s")


def _fill_zeros(ref, nrows, width):
    g = width // 16

    def body(k, _):
        i = k // g
        j = k % g
        ref[i, pl.ds(j * 16, 16)] = jnp.zeros((16,), jnp.float32)
        return 0

    lax.fori_loop(0, nrows * g, body, 0)


IDX_BATCH = 8


def _zero_table(zsrc, table, r0, sem):
    descs = [
        pltpu.async_copy(zsrc, table.at[pl.ds(r0 + k * 128, 128)], sem)
        for k in range(4)
    ]
    descs.append(
        pltpu.async_copy(zsrc.at[pl.ds(0, ZSTRIPE - 512)],
                         table.at[pl.ds(r0 + 512, ZSTRIPE - 512)], sem))
    for d in descs:
        d.wait()


def _copy_out(table, out, sr, o0, sem):
    descs = [
        pltpu.async_copy(table.at[pl.ds(sr + k * 128, 128)],
                         out.at[pl.ds(o0 + k * 128, 128)], sem)
        for k in range(4)
    ]
    descs.append(
        pltpu.async_copy(table.at[pl.ds(sr + 512, ZSTRIPE - 512)],
                         out.at[pl.ds(o0 + 512, ZSTRIPE - 512)], sem))
    for d in descs:
        d.wait()


def _edge_pipeline(feat, src_hbm, dst_hbm, table_sh, row0, nrows,
                   src_a, dst_a, src_b, dst_b, rows0, rows1,
                   sem_g0, sem_g1, sem_s, sem_i):
    npair = nrows // IDX_BATCH // 2
    bufs = (rows0, rows1)
    sems = (sem_g0, sem_g1)

    pltpu.sync_copy(src_hbm.at[pl.ds(row0, IDX_BATCH)], src_a)
    pltpu.sync_copy(dst_hbm.at[pl.ds(row0, IDX_BATCH)], dst_a)
    pltpu.async_copy(feat.at[src_a.at[0]], rows0, sem_g0)
    pltpu.async_copy(feat.at[src_a.at[1]], rows1, sem_g1)

    def proc(cur_src, cur_dst, nxt_src, nxt_dst):
        for j in range(IDX_BATCH):
            buf, sem = bufs[j % 2], sems[j % 2]
            pltpu.make_async_copy(feat.at[pl.ds(0, 128)], buf, sem).wait()
            pltpu.async_copy(buf, table_sh.at[cur_dst.at[j]], sem_s,
                             add=True).wait()
            if j == IDX_BATCH - 3:
                pltpu.make_async_copy(src_hbm.at[pl.ds(0, IDX_BATCH)],
                                      nxt_src, sem_i).wait()
                pltpu.make_async_copy(dst_hbm.at[pl.ds(0, IDX_BATCH)],
                                      nxt_dst, sem_i).wait()
            if j < IDX_BATCH - 2:
                pltpu.async_copy(feat.at[cur_src.at[j + 2]], buf, sem)
            else:
                pltpu.async_copy(feat.at[nxt_src.at[j + 2 - IDX_BATCH]],
                                 buf, sem)

    def pair(p, _):
        base1 = row0 + (2 * p + 1) * IDX_BATCH
        base2 = jnp.minimum(base1 + IDX_BATCH, E_ROWS - IDX_BATCH)
        pltpu.async_copy(src_hbm.at[pl.ds(base1, IDX_BATCH)], src_b, sem_i)
        pltpu.async_copy(dst_hbm.at[pl.ds(base1, IDX_BATCH)], dst_b, sem_i)
        proc(src_a, dst_a, src_b, dst_b)
        pltpu.async_copy(src_hbm.at[pl.ds(base2, IDX_BATCH)], src_a, sem_i)
        pltpu.async_copy(dst_hbm.at[pl.ds(base2, IDX_BATCH)], dst_a, sem_i)
        proc(src_b, dst_b, src_a, dst_a)
        return 0

    lax.fori_loop(0, npair, pair, 0)
    pltpu.make_async_copy(feat.at[pl.ds(0, 128)], rows0, sem_g0).wait()
    pltpu.make_async_copy(feat.at[pl.ds(0, 128)], rows1, sem_g1).wait()


_AGG_SCRATCH = [
    pltpu.VMEM_SHARED((N_TAB, 128), jnp.float32),
    pltpu.VMEM((IDX_BATCH, 128), jnp.int32),
    pltpu.VMEM((IDX_BATCH, 128), jnp.int32),
    pltpu.VMEM((IDX_BATCH, 128), jnp.int32),
    pltpu.VMEM((IDX_BATCH, 128), jnp.int32),
    pltpu.VMEM((128, 128), jnp.float32),
    pltpu.VMEM((128, 128), jnp.float32),
    pltpu.SemaphoreType.DMA,
    pltpu.SemaphoreType.DMA,
    pltpu.SemaphoreType.DMA,
    pltpu.SemaphoreType.DMA,
]


@functools.partial(
    pl.kernel,
    out_type=jax.ShapeDtypeStruct((2 * N_TAB, 128), jnp.float32),
    mesh=_MESH,
    scratch_types=[
        pltpu.VMEM_SHARED((N_TAB, 128), jnp.float32),
        pltpu.VMEM((IDX_BATCH, 128), jnp.int32),
        pltpu.VMEM((128, 128), jnp.float32),
        pltpu.SemaphoreType.DMA,
    ],
)
def _sc_cnt(dst_hbm, cnt_out, table_sh, dst_v, ones_v, sem_s):
    c = lax.axis_index("c")
    s = lax.axis_index("s")

    _fill_zeros(ones_v, 128, 128)
    r0 = s * ZSTRIPE
    _zero_table(ones_v, table_sh, r0, sem_s)

    def fill_ones(k, _):
        i = k // 8
        j = k % 8
        ones_v[i, pl.ds(j * 16, 16)] = jnp.ones((16,), jnp.float32)
        return 0

    lax.fori_loop(0, 1024, fill_ones, 0)
    plsc.subcore_barrier()

    row0 = c * (E_ROWS // 2) + s * ROWS_L1

    def batch(b, _):
        pltpu.sync_copy(dst_hbm.at[pl.ds(row0 + b * IDX_BATCH, IDX_BATCH)],
                        dst_v)
        descs = [
            pltpu.async_copy(ones_v, table_sh.at[dst_v.at[j]], sem_s,
                             add=True)
            for j in range(IDX_BATCH)
        ]
        for d in descs:
            d.wait()
        return 0

    lax.fori_loop(0, ROWS_L1 // IDX_BATCH, batch, 0)
    plsc.subcore_barrier()
    _copy_out(table_sh, cnt_out, r0, c * N_TAB + r0, sem_s)


@functools.partial(
    pl.kernel,
    out_type=jax.ShapeDtypeStruct((2 * N_TAB, 128), jnp.float32),
    mesh=_MESH,
    scratch_types=list(_AGG_SCRATCH),
)
def _sc_agg_l1(x_hbm, src_hbm, dst_hbm, agg_out,
               table_sh, src_a, dst_a, src_b, dst_b, rows0, rows1,
               sem_g0, sem_g1, sem_s, sem_i):
    c = lax.axis_index("c")
    s = lax.axis_index("s")

    _fill_zeros(rows0, 128, 128)
    r0 = s * ZSTRIPE
    _zero_table(rows0, table_sh, r0, sem_g0)
    plsc.subcore_barrier()

    row0 = c * (E_ROWS // 2) + s * ROWS_L1
    _edge_pipeline(x_hbm, src_hbm, dst_hbm, table_sh, row0, ROWS_L1,
                   src_a, dst_a, src_b, dst_b, rows0, rows1,
                   sem_g0, sem_g1, sem_s, sem_i)
    plsc.subcore_barrier()
    _copy_out(table_sh, agg_out, r0, c * N_TAB + r0, sem_g0)


@functools.partial(
    pl.kernel,
    out_type=jax.ShapeDtypeStruct((2 * N_TAB, 128), jnp.float32),
    mesh=_MESH,
    scratch_types=list(_AGG_SCRATCH),
)
def _sc_agg_l23(feat0_hbm, feat1_hbm, src_hbm, dst_hbm, agg_out,
                table_sh, src_a, dst_a, src_b, dst_b, rows0, rows1,
                sem_g0, sem_g1, sem_s, sem_i):
    c = lax.axis_index("c")
    s = lax.axis_index("s")

    _fill_zeros(rows0, 128, 128)
    r0 = s * ZSTRIPE
    _zero_table(rows0, table_sh, r0, sem_g0)
    plsc.subcore_barrier()

    row0 = s * ROWS_L23

    def run(feat):
        _edge_pipeline(feat, src_hbm, dst_hbm, table_sh, row0, ROWS_L23,
                       src_a, dst_a, src_b, dst_b, rows0, rows1,
                       sem_g0, sem_g1, sem_s, sem_i)

    @pl.when(c == 0)
    def _():
        run(feat0_hbm)

    @pl.when(c == 1)
    def _():
        run(feat1_hbm)

    plsc.subcore_barrier()

    sr = s * ZSTRIPE
    _copy_out(table_sh, agg_out, sr, c * N_TAB + sr, sem_g0)


def _dot_t(a, w):
    return lax.dot_general(a, w, (((1,), (1,)), ((), ())),
                           preferred_element_type=jnp.float32)


_ROWS_BLK = 1000
_N_BLKS = N_NODES // _ROWS_BLK


def _tc1_body(agg0, agg1, cnt0, cnt1, x, wl, bl, wr, h0, h1):
    cnt = cnt0[:, 0:1] + cnt1[:, 0:1]
    inv = 1.0 / jnp.maximum(cnt, 1.0)
    mean = (agg0[...] + agg1[...]) * inv
    out = _dot_t(mean, wl[...]) + _dot_t(x[...], wr[...]) + bl[...]
    nrm = jnp.sqrt(jnp.sum(out * out, axis=1, keepdims=True))
    out = out / jnp.maximum(nrm, 1e-12)
    h0[...] = out[:, :128]
    h1[...] = out[:, 128:]


def _tc23_body(agg0, agg1, cnt0, cnt1, x0, x1, wla, wlb, bl, wra, wrb,
               h0, h1):
    cnt = cnt0[:, 0:1] + cnt1[:, 0:1]
    inv = 1.0 / jnp.maximum(cnt, 1.0)
    out = (_dot_t(agg0[...] * inv, wla[...]) + _dot_t(agg1[...] * inv, wlb[...])
           + _dot_t(x0[...], wra[...]) + _dot_t(x1[...], wrb[...]) + bl[...])
    nrm = jnp.sqrt(jnp.sum(out * out, axis=1, keepdims=True))
    out = out / jnp.maximum(nrm, 1e-12)
    h0[...] = out[:, :128]
    h1[...] = out[:, 128:]


def _tc3_body(agg0, agg1, cnt0, cnt1, x0, x1, wla, wlb, bl, wra, wrb,
              cw1, cb1, cw2, cb2, res, acc):
    i = pl.program_id(0)
    cnt = cnt0[:, 0:1] + cnt1[:, 0:1]
    inv = 1.0 / jnp.maximum(cnt, 1.0)
    out = (_dot_t(agg0[...] * inv, wla[...]) + _dot_t(agg1[...] * inv, wlb[...])
           + _dot_t(x0[...], wra[...]) + _dot_t(x1[...], wrb[...]) + bl[...])
    nrm = jnp.sqrt(jnp.sum(out * out, axis=1, keepdims=True))
    out = out / jnp.maximum(nrm, 1e-12)

    @pl.when(i == 0)
    def _():
        acc[...] = jnp.zeros_like(acc)

    acc[...] += jnp.sum(out, axis=0, keepdims=True)

    @pl.when(i == _N_BLKS - 1)
    def _():
        g = acc[...] * (1.0 / N_NODES)
        z = jnp.maximum(_dot_t(g, cw1[...]) + cb1[...], 0.0)
        res[...] = jnp.sum(z * cw2[...], axis=1, keepdims=True) + cb2[...]


def _row_spec(w):
    return pl.BlockSpec((_ROWS_BLK, w), lambda i: (i, 0))


def _full_spec(r, c):
    return pl.BlockSpec((r, c), lambda i: (0, 0))


def kernel(x, edge_index, Wl1, bl1, Wr1, Wl2, bl2, Wr2, Wl3, bl3, Wr3,
           Cw1, Cb1, Cw2, Cb2):
    src = edge_index[0]
    dst = edge_index[1]
    pad = E_PAD - N_EDGES
    srcp = jnp.concatenate(
        [src, jnp.zeros((pad,), jnp.int32)]).reshape(E_ROWS, 128)
    dstp = jnp.concatenate(
        [dst, jnp.full((pad,), N_NODES, jnp.int32)]).reshape(E_ROWS, 128)

    bl1r = bl1[None, :]
    bl2r = bl2[None, :]
    bl3r = bl3[None, :]
    cb1r = Cb1[None, :]
    cb2r = Cb2[None, :]

    cntp = _sc_cnt(dstp)
    cnt0, cnt1 = cntp[:N_NODES], cntp[N_TAB:N_TAB + N_NODES]

    aggp = _sc_agg_l1(x, srcp, dstp)
    agg0, agg1 = aggp[:N_NODES], aggp[N_TAB:N_TAB + N_NODES]

    h0, h1 = pl.pallas_call(
        _tc1_body,
        grid=(_N_BLKS,),
        in_specs=[
            _row_spec(128), _row_spec(128), _row_spec(128), _row_spec(128),
            _row_spec(128), _full_spec(256, 128), _full_spec(1, 256),
            _full_spec(256, 128),
        ],
        out_specs=[_row_spec(128), _row_spec(128)],
        out_shape=[jax.ShapeDtypeStruct((N_NODES, 128), jnp.float32)] * 2,
    )(agg0, agg1, cnt0, cnt1, x, Wl1, bl1r, Wr1)

    aggp = _sc_agg_l23(h0, h1, srcp, dstp)
    agg0, agg1 = aggp[:N_NODES], aggp[N_TAB:N_TAB + N_NODES]
    h0, h1 = pl.pallas_call(
        _tc23_body,
        grid=(_N_BLKS,),
        in_specs=[
            _row_spec(128), _row_spec(128), _row_spec(128), _row_spec(128),
            _row_spec(128), _row_spec(128),
            _full_spec(256, 128), _full_spec(256, 128), _full_spec(1, 256),
            _full_spec(256, 128), _full_spec(256, 128),
        ],
        out_specs=[_row_spec(128), _row_spec(128)],
        out_shape=[jax.ShapeDtypeStruct((N_NODES, 128), jnp.float32)] * 2,
    )(agg0, agg1, cnt0, cnt1, h0, h1,
      Wl2[:, :128], Wl2[:, 128:], bl2r, Wr2[:, :128], Wr2[:, 128:])

    aggp = _sc_agg_l23(h0, h1, srcp, dstp)
    agg0, agg1 = aggp[:N_NODES], aggp[N_TAB:N_TAB + N_NODES]
    res = pl.pallas_call(
        _tc3_body,
        grid=(_N_BLKS,),
        in_specs=[
            _row_spec(128), _row_spec(128), _row_spec(128), _row_spec(128),
            _row_spec(128), _row_spec(128),
            _full_spec(256, 128), _full_spec(256, 128), _full_spec(1, 256),
            _full_spec(256, 128), _full_spec(256, 128),
            _full_spec(256, 256), _full_spec(1, 256), _full_spec(1, 256),
            _full_spec(1, 1),
        ],
        out_specs=pl.BlockSpec((1, 1), lambda i: (0, 0)),
        out_shape=jax.ShapeDtypeStruct((1, 1), jnp.float32),
        scratch_shapes=[pltpu.VMEM((1, 256), jnp.float32)],
        compiler_params=pltpu.CompilerParams(
            dimension_semantics=("arbitrary",)),
    )(agg0, agg1, cnt0, cnt1, h0, h1,
      Wl3[:, :128], Wl3[:, 128:], bl3r, Wr3[:, :128], Wr3[:, 128:],
      Cw1, cb1r, Cw2, cb2r)
    return res
